# speedup vs baseline: 1.4690x; 1.4690x over previous
import functools

import jax
import jax.numpy as jnp
from jax import lax
from jax.experimental import pallas as pl
from jax.experimental.pallas import tpu as pltpu
from jax.experimental.pallas import tpu_sc as plsc

_NC = 2
_NS = 16
_NW = _NC * _NS
_LANE = 16


@functools.lru_cache(maxsize=None)
def _build(b_sz, seq, vocab, d_model):
    PW = seq // _NW
    C = 32
    NCH = (b_sz * PW) // C
    NBUF = 3
    VECS = d_model // _LANE
    HALF = PW // C

    mesh = plsc.VectorSubcoreMesh(core_axis_name="c", subcore_axis_name="s")

    @functools.partial(
        pl.kernel,
        mesh=mesh,
        out_type=jax.ShapeDtypeStruct((b_sz * seq, d_model), jnp.float32),
        scratch_types=[
            pltpu.VMEM((b_sz * PW,), jnp.int32),
            pltpu.VMEM((PW, d_model), jnp.float32),
        ]
        + [pltpu.VMEM((C, d_model), jnp.float32) for _ in range(NBUF)]
        + [pltpu.SemaphoreType.DMA for _ in range(1 + 2 * NBUF)],
    )
    def emb(tok_hbm, wte_hbm, wpe_hbm, out_hbm,
            idx_v, wpe_v, rb0, rb1, rb2,
            s_w, s_g0, s_g1, s_g2, s_o0, s_o1, s_o2):
        rbs = (rb0, rb1, rb2)
        sgs = (s_g0, s_g1, s_g2)
        sos = (s_o0, s_o1, s_o2)
        wid = lax.axis_index("s") * _NC + lax.axis_index("c")
        pos0 = wid * PW

        wpe_cp = pltpu.async_copy(wpe_hbm.at[pl.ds(pos0, PW)], wpe_v, s_w)
        for b in range(b_sz):
            pltpu.sync_copy(tok_hbm.at[pl.ds(b * seq + pos0, PW)],
                            idx_v.at[pl.ds(b * PW, PW)])

        def start_gather(c):
            k = c % NBUF
            return pltpu.async_copy(
                wte_hbm.at[idx_v.at[pl.ds(c * C, C)]], rbs[k], sgs[k])

        gathers = {}
        scatters = {}
        for c in range(min(2, NCH)):
            gathers[c] = start_gather(c)
        wpe_cp.wait()

        for c in range(NCH):
            k = c % NBUF
            h = c % HALF
            gathers[c].wait()
            if c + 2 < NCH:
                if c >= 1:
                    scatters[c - 1].wait()
                gathers[c + 2] = start_gather(c + 2)

            def add_row(i, carry, _rb=rbs[k], _h=h):
                for j in range(VECS):
                    w = wpe_v[_h * C + i, pl.ds(j * _LANE, _LANE)]
                    plsc.addupdate(_rb.at[i, pl.ds(j * _LANE, _LANE)], w)
                return carry
            lax.fori_loop(0, C, add_row, 0)

            row0 = (c // HALF) * seq + pos0 + h * C
            scatters[c] = pltpu.async_copy(
                rbs[k], out_hbm.at[pl.ds(row0, C)], sos[k])

        for c in range(max(0, NCH - 3), NCH):
            scatters[c].wait()

    return emb


def kernel(token_ids, wte, wpe):
    b_sz, seq = token_ids.shape
    vocab, d_model = wte.shape
    emb = _build(b_sz, seq, vocab, d_model)
    out = emb(token_ids.reshape(-1), wte, wpe)
    return out.reshape(b_sz, seq, d_model)


# speedup vs baseline: 1.4698x; 1.0005x over previous
import functools

import jax
import jax.numpy as jnp
from jax import lax
from jax.experimental import pallas as pl
from jax.experimental.pallas import tpu as pltpu
from jax.experimental.pallas import tpu_sc as plsc

_NC = 2
_NS = 16
_NW = _NC * _NS
_LANE = 16


@functools.lru_cache(maxsize=None)
def _build(b_sz, seq, vocab, d_model):
    PW = seq // _NW
    C = 16
    NCH = (b_sz * PW) // C
    NBUF = 6
    LOOKAHEAD = 3
    VECS = d_model // _LANE
    QPB = PW // C

    mesh = plsc.VectorSubcoreMesh(core_axis_name="c", subcore_axis_name="s")

    @functools.partial(
        pl.kernel,
        mesh=mesh,
        out_type=jax.ShapeDtypeStruct((b_sz * seq, d_model), jnp.float32),
        scratch_types=[
            pltpu.VMEM((b_sz * PW,), jnp.int32),
            pltpu.VMEM((PW, d_model), jnp.float32),
        ]
        + [pltpu.VMEM((C, d_model), jnp.float32) for _ in range(NBUF)]
        + [pltpu.SemaphoreType.DMA for _ in range(1 + 2 * NBUF)],
    )
    def emb(tok_hbm, wte_hbm, wpe_hbm, out_hbm,
            idx_v, wpe_v, rb0, rb1, rb2, rb3, rb4, rb5,
            s_w, s_g0, s_g1, s_g2, s_g3, s_g4, s_g5,
            s_o0, s_o1, s_o2, s_o3, s_o4, s_o5):
        rbs = (rb0, rb1, rb2, rb3, rb4, rb5)
        sgs = (s_g0, s_g1, s_g2, s_g3, s_g4, s_g5)
        sos = (s_o0, s_o1, s_o2, s_o3, s_o4, s_o5)
        wid = lax.axis_index("s") * _NC + lax.axis_index("c")
        pos0 = wid * PW

        wpe_cp = pltpu.async_copy(wpe_hbm.at[pl.ds(pos0, PW)], wpe_v, s_w)
        for b in range(b_sz):
            pltpu.sync_copy(tok_hbm.at[pl.ds(b * seq + pos0, PW)],
                            idx_v.at[pl.ds(b * PW, PW)])

        def start_gather(c):
            k = c % NBUF
            return pltpu.async_copy(
                wte_hbm.at[idx_v.at[pl.ds(c * C, C)]], rbs[k], sgs[k])

        gathers = {}
        scatters = {}
        for c in range(min(LOOKAHEAD, NCH)):
            gathers[c] = start_gather(c)
        wpe_cp.wait()

        for c in range(NCH):
            k = c % NBUF
            q = c % QPB
            n = c + LOOKAHEAD
            if n < NCH:
                if n >= NBUF:
                    scatters[n - NBUF].wait()
                gathers[n] = start_gather(n)
            gathers[c].wait()

            def add_row(i, carry, _rb=rbs[k], _q=q):
                for j in range(VECS):
                    w = wpe_v[_q * C + i, pl.ds(j * _LANE, _LANE)]
                    plsc.addupdate(_rb.at[i, pl.ds(j * _LANE, _LANE)], w)
                return carry
            lax.fori_loop(0, C, add_row, 0)

            row0 = (c // QPB) * seq + pos0 + q * C
            scatters[c] = pltpu.async_copy(
                rbs[k], out_hbm.at[pl.ds(row0, C)], sos[k])

        for c in range(max(0, NCH - NBUF), NCH):
            scatters[c].wait()

    return emb


def kernel(token_ids, wte, wpe):
    b_sz, seq = token_ids.shape
    vocab, d_model = wte.shape
    emb = _build(b_sz, seq, vocab, d_model)
    out = emb(token_ids.reshape(-1), wte, wpe)
    return out.reshape(b_sz, seq, d_model)


# speedup vs baseline: 1.7619x; 1.1987x over previous
import functools

import jax
import jax.numpy as jnp
from jax import lax
from jax.experimental import pallas as pl
from jax.experimental.pallas import tpu as pltpu
from jax.experimental.pallas import tpu_sc as plsc

_NC = 2
_NS = 16
_NW = _NC * _NS
_LANE = 16


@functools.lru_cache(maxsize=None)
def _build(b_sz, seq, vocab, d_model):
    PW = seq // _NW
    P = 8
    NQ = PW // P
    NBUF = 3
    LOOKAHEAD = 2
    VECS = d_model // _LANE

    mesh = plsc.VectorSubcoreMesh(core_axis_name="c", subcore_axis_name="s")

    @functools.partial(
        pl.kernel,
        mesh=mesh,
        out_type=jax.ShapeDtypeStruct((b_sz * seq, d_model), jnp.float32),
        scratch_types=[
            pltpu.VMEM((b_sz * PW,), jnp.int32),
            pltpu.VMEM((PW, d_model), jnp.float32),
        ]
        + [pltpu.VMEM((b_sz * P, d_model), jnp.float32) for _ in range(NBUF)]
        + [pltpu.SemaphoreType.DMA for _ in range(1 + 2 * NBUF)],
    )
    def emb(tok_hbm, wte_hbm, wpe_hbm, out_hbm,
            idx_v, wpe_v, rb0, rb1, rb2,
            s_w, s_g0, s_g1, s_g2, s_o0, s_o1, s_o2):
        rbs = (rb0, rb1, rb2)
        sgs = (s_g0, s_g1, s_g2)
        sos = (s_o0, s_o1, s_o2)
        wid = lax.axis_index("s") * _NC + lax.axis_index("c")
        pos0 = wid * PW

        wpe_cp = pltpu.async_copy(wpe_hbm.at[pl.ds(pos0, PW)], wpe_v, s_w)
        for b in range(b_sz):
            pltpu.sync_copy(tok_hbm.at[pl.ds(b * seq + pos0, PW)],
                            idx_v.at[pl.ds(b * PW, PW)])

        def start_gathers(q):
            k = q % NBUF
            return [
                pltpu.async_copy(
                    wte_hbm.at[idx_v.at[pl.ds(b * PW + q * P, P)]],
                    rbs[k].at[pl.ds(b * P, P)], sgs[k])
                for b in range(b_sz)
            ]

        gathers = {}
        scatters = {}
        for q in range(min(LOOKAHEAD, NQ)):
            gathers[q] = start_gathers(q)
        wpe_cp.wait()

        for q in range(NQ):
            k = q % NBUF
            n = q + LOOKAHEAD
            if n < NQ:
                if n >= NBUF:
                    for cp in scatters[n - NBUF]:
                        cp.wait()
                gathers[n] = start_gathers(n)
            for cp in gathers[q]:
                cp.wait()

            def add_pos(i, carry, _rb=rbs[k], _q=q):
                for j in range(VECS):
                    w = wpe_v[_q * P + i, pl.ds(j * _LANE, _LANE)]
                    for b in range(b_sz):
                        plsc.addupdate(
                            _rb.at[b * P + i, pl.ds(j * _LANE, _LANE)], w)
                return carry
            lax.fori_loop(0, P, add_pos, 0)

            scatters[q] = [
                pltpu.async_copy(
                    rbs[k].at[pl.ds(b * P, P)],
                    out_hbm.at[pl.ds(b * seq + pos0 + q * P, P)], sos[k])
                for b in range(b_sz)
            ]

        for q in range(max(0, NQ - NBUF), NQ):
            for cp in scatters[q]:
                cp.wait()

    return emb


def kernel(token_ids, wte, wpe):
    b_sz, seq = token_ids.shape
    vocab, d_model = wte.shape
    emb = _build(b_sz, seq, vocab, d_model)
    out = emb(token_ids.reshape(-1), wte, wpe)
    return out.reshape(b_sz, seq, d_model)


# speedup vs baseline: 1.7662x; 1.0025x over previous
import functools

import jax
import jax.numpy as jnp
from jax import lax
from jax.experimental import pallas as pl
from jax.experimental.pallas import tpu as pltpu
from jax.experimental.pallas import tpu_sc as plsc

_NC = 2
_NS = 16
_NW = _NC * _NS
_LANE = 16


@functools.lru_cache(maxsize=None)
def _build(b_sz, seq, vocab, d_model):
    PW = seq // _NW
    P = 8
    NQ = PW // P
    NBUF = 3
    LOOKAHEAD = 2
    VECS = d_model // _LANE

    mesh = plsc.VectorSubcoreMesh(core_axis_name="c", subcore_axis_name="s")

    @functools.partial(
        pl.kernel,
        mesh=mesh,
        out_type=jax.ShapeDtypeStruct((b_sz, seq, d_model), jnp.float32),
        scratch_types=[
            pltpu.VMEM((b_sz * PW,), jnp.int32),
            pltpu.VMEM((PW, d_model), jnp.float32),
        ]
        + [pltpu.VMEM((b_sz * P, d_model), jnp.float32) for _ in range(NBUF)]
        + [pltpu.SemaphoreType.DMA for _ in range(1 + 2 * NBUF)],
    )
    def emb(tok_hbm, wte_hbm, wpe_hbm, out_hbm,
            idx_v, wpe_v, rb0, rb1, rb2,
            s_w, s_g0, s_g1, s_g2, s_o0, s_o1, s_o2):
        rbs = (rb0, rb1, rb2)
        sgs = (s_g0, s_g1, s_g2)
        sos = (s_o0, s_o1, s_o2)
        wid = lax.axis_index("s") * _NC + lax.axis_index("c")
        pos0 = wid * PW

        wpe_cp = pltpu.async_copy(wpe_hbm.at[pl.ds(pos0, PW)], wpe_v, s_w)
        for b in range(b_sz):
            pltpu.sync_copy(tok_hbm.at[b, pl.ds(pos0, PW)],
                            idx_v.at[pl.ds(b * PW, PW)])

        def start_gathers(q):
            k = q % NBUF
            return [
                pltpu.async_copy(
                    wte_hbm.at[idx_v.at[pl.ds(b * PW + q * P, P)]],
                    rbs[k].at[pl.ds(b * P, P)], sgs[k])
                for b in range(b_sz)
            ]

        gathers = {}
        scatters = {}
        for q in range(min(LOOKAHEAD, NQ)):
            gathers[q] = start_gathers(q)
        wpe_cp.wait()

        for q in range(NQ):
            k = q % NBUF
            n = q + LOOKAHEAD
            if n < NQ:
                if n >= NBUF:
                    for cp in scatters[n - NBUF]:
                        cp.wait()
                gathers[n] = start_gathers(n)
            for cp in gathers[q]:
                cp.wait()

            def add_pos(i, carry, _rb=rbs[k], _q=q):
                for j in range(VECS):
                    w = wpe_v[_q * P + i, pl.ds(j * _LANE, _LANE)]
                    for b in range(b_sz):
                        plsc.addupdate(
                            _rb.at[b * P + i, pl.ds(j * _LANE, _LANE)], w)
                return carry
            lax.fori_loop(0, P, add_pos, 0)

            scatters[q] = [
                pltpu.async_copy(
                    rbs[k].at[pl.ds(b * P, P)],
                    out_hbm.at[b, pl.ds(pos0 + q * P, P)], sos[k])
                for b in range(b_sz)
            ]

        for q in range(max(0, NQ - NBUF), NQ):
            for cp in scatters[q]:
                cp.wait()

    return emb


def kernel(token_ids, wte, wpe):
    b_sz, seq = token_ids.shape
    vocab, d_model = wte.shape
    emb = _build(b_sz, seq, vocab, d_model)
    return emb(token_ids, wte, wpe)


# speedup vs baseline: 1.8020x; 1.0202x over previous
import functools

import jax
import jax.numpy as jnp
from jax import lax
from jax.experimental import pallas as pl
from jax.experimental.pallas import tpu as pltpu
from jax.experimental.pallas import tpu_sc as plsc

_NC = 2
_NS = 16
_NW = _NC * _NS
_LANE = 16


@functools.lru_cache(maxsize=None)
def _build(b_sz, seq, vocab, d_model):
    PW = seq // _NW
    P = 8
    NQ = PW // P
    NBUF = 3
    LOOKAHEAD = 2
    VECS = d_model // _LANE

    mesh = plsc.VectorSubcoreMesh(core_axis_name="c", subcore_axis_name="s")

    @functools.partial(
        pl.kernel,
        mesh=mesh,
        out_type=jax.ShapeDtypeStruct((b_sz, seq, d_model), jnp.float32),
        scratch_types=[
            pltpu.VMEM((b_sz, PW), jnp.int32),
            pltpu.VMEM((PW, d_model), jnp.float32),
        ]
        + [pltpu.VMEM((b_sz * P, d_model), jnp.float32) for _ in range(NBUF)]
        + [pltpu.SemaphoreType.DMA for _ in range(2 + 2 * NBUF)],
    )
    def emb(tok_hbm, wte_hbm, wpe_hbm, out_hbm,
            idx_v, wpe_v, rb0, rb1, rb2,
            s_w, s_i, s_g0, s_g1, s_g2, s_o0, s_o1, s_o2):
        rbs = (rb0, rb1, rb2)
        sgs = (s_g0, s_g1, s_g2)
        sos = (s_o0, s_o1, s_o2)
        wid = lax.axis_index("s") * _NC + lax.axis_index("c")
        pos0 = wid * PW

        idx_cps = [
            pltpu.async_copy(tok_hbm.at[b, pl.ds(pos0, PW)],
                             idx_v.at[b], s_i)
            for b in range(b_sz)
        ]
        wpe_cp = pltpu.async_copy(wpe_hbm.at[pl.ds(pos0, PW)], wpe_v, s_w)

        def start_gathers(q):
            k = q % NBUF
            return [
                pltpu.async_copy(
                    wte_hbm.at[idx_v.at[b, pl.ds(q * P, P)]],
                    rbs[k].at[pl.ds(b * P, P)], sgs[k])
                for b in range(b_sz)
            ]

        gathers = {}
        scatters = {}
        for cp in idx_cps:
            cp.wait()
        for q in range(min(LOOKAHEAD, NQ)):
            gathers[q] = start_gathers(q)
        wpe_cp.wait()

        for q in range(NQ):
            k = q % NBUF
            n = q + LOOKAHEAD
            if n < NQ:
                if n >= NBUF:
                    for cp in scatters[n - NBUF]:
                        cp.wait()
                gathers[n] = start_gathers(n)
            for cp in gathers[q]:
                cp.wait()

            def add_pos(i, carry, _rb=rbs[k], _q=q):
                for j in range(VECS):
                    w = wpe_v[_q * P + i, pl.ds(j * _LANE, _LANE)]
                    for b in range(b_sz):
                        plsc.addupdate(
                            _rb.at[b * P + i, pl.ds(j * _LANE, _LANE)], w)
                return carry
            lax.fori_loop(0, P, add_pos, 0)

            scatters[q] = [
                pltpu.async_copy(
                    rbs[k].at[pl.ds(b * P, P)],
                    out_hbm.at[b, pl.ds(pos0 + q * P, P)], sos[k])
                for b in range(b_sz)
            ]

        for q in range(max(0, NQ - NBUF), NQ):
            for cp in scatters[q]:
                cp.wait()

    return emb


def kernel(token_ids, wte, wpe):
    b_sz, seq = token_ids.shape
    vocab, d_model = wte.shape
    emb = _build(b_sz, seq, vocab, d_model)
    return emb(token_ids, wte, wpe)
